# baseline (speedup 1.0000x reference)
import jax
import jax.numpy as jnp
from jax.experimental import pallas as pl
from jax.experimental.pallas import tpu as pltpu


def _pos_kernel(col_ref, row_ref, out_ref):
    _, h, w, d2 = out_ref.shape
    d = d2 // 2
    out_ref[0, :, :, :d] = jnp.broadcast_to(
        col_ref[...][None, :, :], (h, w, d))
    out_ref[0, :, :, d:] = jnp.broadcast_to(
        row_ref[...][:, None, :], (h, w, d))


def kernel(x, row_embed, col_embed):
    b = x.shape[0]
    h, w = x.shape[-2], x.shape[-1]
    d = row_embed.shape[1]
    out = pl.pallas_call(
        _pos_kernel,
        grid=(b,),
        in_specs=[
            pl.BlockSpec((w, d), lambda i: (0, 0)),
            pl.BlockSpec((h, d), lambda i: (0, 0)),
        ],
        out_specs=pl.BlockSpec((1, h, w, 2 * d), lambda i: (i, 0, 0, 0)),
        out_shape=jax.ShapeDtypeStruct((b, h, w, 2 * d), jnp.float32),
    )(col_embed[:w], row_embed[:h])
    return jnp.transpose(out, (0, 3, 1, 2))
